# baseline (speedup 1.0000x reference)
import functools

import jax
import jax.numpy as jnp
from jax import lax
from jax.experimental import pallas as pl
from jax.experimental.pallas import tpu as pltpu
from jax.experimental.pallas import tpu_sc as plsc

_INF_BITS = 0x7F800000


def _mine_kernel(emb_ref, labr_ref, labc_ref, mm_ref, d2ap_ref, enc_ref,
                 embn_ref, embt_ref, *, blk_r, b, row_off):
    i = pl.program_id(0)

    @pl.when(i == 0)
    def _init():
        e = emb_ref[...]
        nrm = jnp.sqrt(jnp.sum(e * e, axis=1, keepdims=True))
        en = e / jnp.maximum(nrm, 1e-12)
        embn_ref[...] = en
        embt_ref[...] = en.T

    r0 = row_off + i * blk_r
    rows = embn_ref[pl.ds(r0, blk_r), :]
    ent = embt_ref[...]
    g = jnp.dot(rows, ent, preferred_element_type=jnp.float32)
    sq_cols = jnp.sum(ent * ent, axis=0, keepdims=True)
    sq_rows = jnp.sum(rows * rows, axis=1, keepdims=True)
    d2 = sq_rows + sq_cols - 2.0 * g

    lab_cols = labr_ref[...]
    lab_rows = labc_ref[pl.ds(r0, blk_r), :]
    same = lab_rows == lab_cols
    col_ids = lax.broadcasted_iota(jnp.int32, (blk_r, b), 1)
    row_ids = lax.broadcasted_iota(jnp.int32, (blk_r, b), 0) + r0

    neg_inf = jnp.float32(-jnp.inf)
    pos_mask = same & (col_ids != row_ids)
    d2_ap = jnp.max(jnp.where(pos_mask, d2, neg_inf), axis=1, keepdims=True)
    d2_ap_c = jnp.maximum(d2_ap, 0.0)
    d_ap = jnp.sqrt(d2_ap_c)

    n_lab = mm_ref.shape[0]
    oh_rows = (lab_rows == lax.broadcasted_iota(jnp.int32, (1, n_lab), 1)
               ).astype(jnp.float32)
    oh_cols = (lax.broadcasted_iota(jnp.int32, (n_lab, 1), 0) == lab_cols
               ).astype(jnp.float32)
    mrow = jnp.dot(oh_rows, mm_ref[...], preferred_element_type=jnp.float32)
    margins = jnp.dot(mrow, oh_cols, preferred_element_type=jnp.float32)

    thr = d_ap + margins
    semi = (~same) & (d2 > d2_ap_c) & (d2 < thr * thr)

    enc = (lax.bitcast_convert_type(d2, jnp.int32) & jnp.int32(~7)) | lab_cols
    enc_min = jnp.min(jnp.where(semi, enc, jnp.int32(_INF_BITS)),
                      axis=1, keepdims=True)

    d2ap_ref[pl.ds(i * blk_r, blk_r), :] = d2_ap
    enc_ref[pl.ds(i * blk_r, blk_r), :] = enc_min


def _newton_sqrt(x):
    bits = lax.bitcast_convert_type(x, jnp.int32)
    y = lax.bitcast_convert_type(
        jnp.int32(0x5F3759DF) - lax.shift_right_arithmetic(bits, 1),
        jnp.float32)
    for _ in range(3):
        y = y * (1.5 - 0.5 * x * y * y)
    return x * y


def _make_sc_stage(n_loc, n_workers):
    apw = n_loc // n_workers
    n_chunks = apw // 16
    mesh = plsc.VectorSubcoreMesh(core_axis_name="c", subcore_axis_name="s",
                                  num_cores=2, num_subcores=16)

    @functools.partial(
        pl.kernel, mesh=mesh,
        out_type=jax.ShapeDtypeStruct((2 * n_workers, 16), jnp.float32),
        scratch_types=[
            pltpu.VMEM((apw,), jnp.float32),
            pltpu.VMEM((apw,), jnp.int32),
            pltpu.VMEM((apw,), jnp.int32),
            pltpu.VMEM((64,), jnp.float32),
            pltpu.VMEM((16,), jnp.float32),
            pltpu.VMEM((16,), jnp.float32),
        ],
    )
    def sc_stage(d2ap_hbm, enc_hbm, lab_hbm, mm_hbm, out_hbm,
                 d2ap_v, enc_v, lab_v, mm_v, psum_v, pcnt_v):
        c = lax.axis_index("c")
        s = lax.axis_index("s")
        wid = s * 2 + c
        base = wid * apw
        pltpu.sync_copy(d2ap_hbm.at[pl.ds(base, apw)], d2ap_v)
        pltpu.sync_copy(enc_hbm.at[pl.ds(base, apw)], enc_v)
        pltpu.sync_copy(lab_hbm.at[pl.ds(base, apw)], lab_v)
        pltpu.sync_copy(mm_hbm, mm_v)

        t0 = mm_v[pl.ds(0, 16)]
        t1 = mm_v[pl.ds(16, 16)]
        t2 = mm_v[pl.ds(32, 16)]
        t3 = mm_v[pl.ds(48, 16)]

        psum = jnp.zeros((16,), jnp.float32)
        pcnt = jnp.zeros((16,), jnp.float32)
        for k in range(n_chunks):
            sl = pl.ds(k * 16, 16)
            d2ap_raw = d2ap_v[sl]
            enc = enc_v[sl]
            la = lab_v[sl]
            has_pos = d2ap_raw > jnp.float32(-jnp.inf)
            has_neg = enc < jnp.int32(_INF_BITS)
            lab_n = enc & jnp.int32(7)
            d2an = lax.bitcast_convert_type(enc & jnp.int32(~7), jnp.float32)
            d_ap = _newton_sqrt(jnp.maximum(d2ap_raw, 0.0))
            d_an = _newton_sqrt(d2an)
            midx = la * 8 + lab_n
            mlo = midx & jnp.int32(15)
            g0 = t0.at[mlo].get(mode="promise_in_bounds")
            g1 = t1.at[mlo].get(mode="promise_in_bounds")
            g2 = t2.at[mlo].get(mode="promise_in_bounds")
            g3 = t3.at[mlo].get(mode="promise_in_bounds")
            m = jnp.where(midx < 16, g0,
                          jnp.where(midx < 32, g1,
                                    jnp.where(midx < 48, g2, g3)))
            valid = has_pos & has_neg
            loss = jnp.maximum(d_ap - d_an + m, 0.0)
            psum = psum + jnp.where(valid, loss, 0.0)
            pcnt = pcnt + jnp.where(valid, 1.0, 0.0)
        psum_v[...] = psum
        pcnt_v[...] = pcnt
        pltpu.sync_copy(psum_v, out_hbm.at[wid])
        pltpu.sync_copy(pcnt_v, out_hbm.at[n_workers + wid])

    return sc_stage


def _finish_kernel(p0_ref, p1_ref, out_ref, *, n_workers):
    x0 = p0_ref[...]
    x1 = p1_ref[...]
    total = jnp.sum(x0[0:n_workers, :]) + jnp.sum(x1[0:n_workers, :])
    cnt = (jnp.sum(x0[n_workers:2 * n_workers, :])
           + jnp.sum(x1[n_workers:2 * n_workers, :]))
    out_ref[0, 0] = jnp.where(cnt > 0.0, total / jnp.maximum(cnt, 1.0), 0.0)


def kernel(embeddings, labels, margin_matrix):
    b, d = embeddings.shape
    blk_r = 512
    n_half = 2
    half = b // n_half
    n_workers = 32
    lab_row = labels.reshape(1, b)
    lab_col = labels.reshape(b, 1)
    n_lab = margin_matrix.shape[0]
    mm_flat = margin_matrix.reshape(n_lab * n_lab)
    sc_stage = _make_sc_stage(half, n_workers)

    mine = lambda row_off: pl.pallas_call(
        functools.partial(_mine_kernel, blk_r=blk_r, b=b, row_off=row_off),
        grid=(half // blk_r,),
        in_specs=[
            pl.BlockSpec((b, d), lambda i: (0, 0)),
            pl.BlockSpec((1, b), lambda i: (0, 0)),
            pl.BlockSpec((b, 1), lambda i: (0, 0)),
            pl.BlockSpec((n_lab, n_lab), lambda i: (0, 0)),
        ],
        out_specs=[
            pl.BlockSpec((half, 1), lambda i: (0, 0)),
            pl.BlockSpec((half, 1), lambda i: (0, 0)),
        ],
        out_shape=[
            jax.ShapeDtypeStruct((half, 1), jnp.float32),
            jax.ShapeDtypeStruct((half, 1), jnp.int32),
        ],
        scratch_shapes=[
            pltpu.VMEM((b, d), jnp.float32),
            pltpu.VMEM((d, b), jnp.float32),
        ],
    )(embeddings, lab_row, lab_col, margin_matrix)

    partials = []
    for h in range(n_half):
        d2ap, enc = mine(h * half)
        partials.append(sc_stage(d2ap.reshape(half), enc.reshape(half),
                                 labels[h * half:(h + 1) * half], mm_flat))

    out = pl.pallas_call(
        functools.partial(_finish_kernel, n_workers=n_workers),
        out_specs=pl.BlockSpec(memory_space=pltpu.SMEM),
        out_shape=jax.ShapeDtypeStruct((1, 1), jnp.float32),
    )(*partials)
    return out[0, 0]


# speedup vs baseline: 1.2416x; 1.2416x over previous
import functools

import jax
import jax.numpy as jnp
from jax import lax
from jax.experimental import pallas as pl
from jax.experimental.pallas import tpu as pltpu
from jax.experimental.pallas import tpu_sc as plsc

_INF_BITS = 0x7F800000


def _mine_kernel(emb_ref, labr_ref, labc_ref, mm_ref, d2ap_ref, enc_ref,
                 embn_ref, embt_ref, *, blk_r, b):
    i = pl.program_id(0)

    @pl.when(i == 0)
    def _init():
        e = emb_ref[...]
        nrm = jnp.sqrt(jnp.sum(e * e, axis=1, keepdims=True))
        en = e / jnp.maximum(nrm, 1e-12)
        embn_ref[...] = en
        embt_ref[...] = en.T

    r0 = i * blk_r
    rows = embn_ref[pl.ds(r0, blk_r), :]
    ent = embt_ref[...]
    g = jnp.dot(rows, ent, preferred_element_type=jnp.float32)
    sq_cols = jnp.sum(ent * ent, axis=0, keepdims=True)
    sq_rows = jnp.sum(rows * rows, axis=1, keepdims=True)
    d2 = sq_rows + sq_cols - 2.0 * g

    lab_cols = labr_ref[...]
    lab_rows = labc_ref[pl.ds(r0, blk_r), :]
    same = lab_rows == lab_cols

    neg_inf = jnp.float32(-jnp.inf)
    d2_ap = jnp.max(jnp.where(same, d2, neg_inf), axis=1, keepdims=True)
    d2_ap_c = jnp.maximum(d2_ap, 0.0)
    d_ap = jnp.sqrt(d2_ap_c)

    n_lab = mm_ref.shape[0]
    oh_rows = (lab_rows == lax.broadcasted_iota(jnp.int32, (1, n_lab), 1)
               ).astype(jnp.float32)
    oh_cols = (lax.broadcasted_iota(jnp.int32, (n_lab, 1), 0) == lab_cols
               ).astype(jnp.float32)
    mrow = jnp.dot(oh_rows, mm_ref[...], preferred_element_type=jnp.float32)
    margins = jnp.dot(mrow, oh_cols, preferred_element_type=jnp.float32)

    cls_sz = jnp.sum(oh_cols, axis=1, keepdims=True)
    row_sz = jnp.dot(oh_rows, cls_sz, preferred_element_type=jnp.float32)
    has_pos = row_sz >= 2.0

    thr = d_ap + margins
    semi = (d2 > d2_ap_c) & (d2 < thr * thr)

    enc = (lax.bitcast_convert_type(d2, jnp.int32) & jnp.int32(~7)) | lab_cols
    enc_min = jnp.min(jnp.where(semi, enc, jnp.int32(_INF_BITS)),
                      axis=1, keepdims=True)

    d2ap_ref[pl.ds(r0, blk_r), :] = jnp.where(has_pos, d2_ap, neg_inf)
    enc_ref[pl.ds(r0, blk_r), :] = enc_min


def _newton_sqrt(x):
    bits = lax.bitcast_convert_type(x, jnp.int32)
    y = lax.bitcast_convert_type(
        jnp.int32(0x5F3759DF) - lax.shift_right_arithmetic(bits, 1),
        jnp.float32)
    for _ in range(3):
        y = y * (1.5 - 0.5 * x * y * y)
    return x * y


def _make_sc_stage(b, n_workers):
    apw = b // n_workers
    n_chunks = apw // 16
    mesh = plsc.VectorSubcoreMesh(core_axis_name="c", subcore_axis_name="s",
                                  num_cores=2, num_subcores=16)

    @functools.partial(
        pl.kernel, mesh=mesh,
        out_type=jax.ShapeDtypeStruct((2 * n_workers, 16), jnp.float32),
        scratch_types=[
            pltpu.VMEM((apw,), jnp.float32),
            pltpu.VMEM((apw,), jnp.int32),
            pltpu.VMEM((apw,), jnp.int32),
            pltpu.VMEM((64,), jnp.float32),
            pltpu.VMEM((16,), jnp.float32),
            pltpu.VMEM((16,), jnp.float32),
        ],
    )
    def sc_stage(d2ap_hbm, enc_hbm, lab_hbm, mm_hbm, out_hbm,
                 d2ap_v, enc_v, lab_v, mm_v, psum_v, pcnt_v):
        c = lax.axis_index("c")
        s = lax.axis_index("s")
        wid = s * 2 + c
        base = wid * apw
        pltpu.sync_copy(d2ap_hbm.at[pl.ds(base, apw)], d2ap_v)
        pltpu.sync_copy(enc_hbm.at[pl.ds(base, apw)], enc_v)
        pltpu.sync_copy(lab_hbm.at[pl.ds(base, apw)], lab_v)
        pltpu.sync_copy(mm_hbm, mm_v)

        t0 = mm_v[pl.ds(0, 16)]
        t1 = mm_v[pl.ds(16, 16)]
        t2 = mm_v[pl.ds(32, 16)]
        t3 = mm_v[pl.ds(48, 16)]

        psum = jnp.zeros((16,), jnp.float32)
        pcnt = jnp.zeros((16,), jnp.float32)
        for k in range(n_chunks):
            sl = pl.ds(k * 16, 16)
            d2ap_raw = d2ap_v[sl]
            enc = enc_v[sl]
            la = lab_v[sl]
            has_pos = d2ap_raw > jnp.float32(-jnp.inf)
            has_neg = enc < jnp.int32(_INF_BITS)
            lab_n = enc & jnp.int32(7)
            d2an = lax.bitcast_convert_type(enc & jnp.int32(~7), jnp.float32)
            d_ap = _newton_sqrt(jnp.maximum(d2ap_raw, 0.0))
            d_an = _newton_sqrt(d2an)
            midx = la * 8 + lab_n
            mlo = midx & jnp.int32(15)
            g0 = t0.at[mlo].get(mode="promise_in_bounds")
            g1 = t1.at[mlo].get(mode="promise_in_bounds")
            g2 = t2.at[mlo].get(mode="promise_in_bounds")
            g3 = t3.at[mlo].get(mode="promise_in_bounds")
            m = jnp.where(midx < 16, g0,
                          jnp.where(midx < 32, g1,
                                    jnp.where(midx < 48, g2, g3)))
            valid = has_pos & has_neg
            loss = jnp.maximum(d_ap - d_an + m, 0.0)
            psum = psum + jnp.where(valid, loss, 0.0)
            pcnt = pcnt + jnp.where(valid, 1.0, 0.0)
        psum_v[...] = psum
        pcnt_v[...] = pcnt
        pltpu.sync_copy(psum_v, out_hbm.at[wid])
        pltpu.sync_copy(pcnt_v, out_hbm.at[n_workers + wid])

    return sc_stage


def _finish_kernel(part_ref, out_ref, *, n_workers):
    x = part_ref[...]
    total = jnp.sum(x[0:n_workers, :])
    cnt = jnp.sum(x[n_workers:2 * n_workers, :])
    out_ref[0, 0] = jnp.where(cnt > 0.0, total / jnp.maximum(cnt, 1.0), 0.0)


def kernel(embeddings, labels, margin_matrix):
    b, d = embeddings.shape
    blk_r = 512
    n_blk = b // blk_r
    n_workers = 32
    lab_row = labels.reshape(1, b)
    lab_col = labels.reshape(b, 1)
    n_lab = margin_matrix.shape[0]
    d2ap, enc = pl.pallas_call(
        functools.partial(_mine_kernel, blk_r=blk_r, b=b),
        grid=(n_blk,),
        in_specs=[
            pl.BlockSpec((b, d), lambda i: (0, 0)),
            pl.BlockSpec((1, b), lambda i: (0, 0)),
            pl.BlockSpec((b, 1), lambda i: (0, 0)),
            pl.BlockSpec((n_lab, n_lab), lambda i: (0, 0)),
        ],
        out_specs=[
            pl.BlockSpec((b, 1), lambda i: (0, 0)),
            pl.BlockSpec((b, 1), lambda i: (0, 0)),
        ],
        out_shape=[
            jax.ShapeDtypeStruct((b, 1), jnp.float32),
            jax.ShapeDtypeStruct((b, 1), jnp.int32),
        ],
        scratch_shapes=[
            pltpu.VMEM((b, d), jnp.float32),
            pltpu.VMEM((d, b), jnp.float32),
        ],
    )(embeddings, lab_row, lab_col, margin_matrix)

    sc_stage = _make_sc_stage(b, n_workers)
    partials = sc_stage(d2ap.reshape(b), enc.reshape(b), labels,
                        margin_matrix.reshape(n_lab * n_lab))

    out = pl.pallas_call(
        functools.partial(_finish_kernel, n_workers=n_workers),
        out_specs=pl.BlockSpec(memory_space=pltpu.SMEM),
        out_shape=jax.ShapeDtypeStruct((1, 1), jnp.float32),
    )(partials)
    return out[0, 0]


# speedup vs baseline: 1.2863x; 1.0360x over previous
import functools

import jax
import jax.numpy as jnp
from jax import lax
from jax.experimental import pallas as pl
from jax.experimental.pallas import tpu as pltpu
from jax.experimental.pallas import tpu_sc as plsc

_INF_BITS = 0x7F800000


def _mine_kernel(emb_ref, labr_ref, labc_ref, mm_ref, d2ap_ref, enc_ref,
                 embn_ref, embt_ref, *, blk_r, b):
    i = pl.program_id(0)

    @pl.when(i == 0)
    def _init():
        e = emb_ref[...]
        nrm = jnp.sqrt(jnp.sum(e * e, axis=1, keepdims=True))
        en = e / jnp.maximum(nrm, 1e-12)
        embn_ref[...] = en
        embt_ref[...] = en.T

    r0 = i * blk_r
    rows = embn_ref[pl.ds(r0, blk_r), :]
    ent = embt_ref[...]
    g = jnp.dot(rows, ent, preferred_element_type=jnp.float32)
    sq_cols = jnp.sum(ent * ent, axis=0, keepdims=True)
    sq_rows = jnp.sum(rows * rows, axis=1, keepdims=True)
    d2 = sq_rows + sq_cols - 2.0 * g

    lab_cols = labr_ref[...]
    lab_rows = labc_ref[pl.ds(r0, blk_r), :]
    same = lab_rows == lab_cols

    neg_inf = jnp.float32(-jnp.inf)
    d2_ap = jnp.max(jnp.where(same, d2, neg_inf), axis=1, keepdims=True)
    d2_ap_c = jnp.maximum(d2_ap, 0.0)
    d_ap = jnp.sqrt(d2_ap_c)

    n_lab = mm_ref.shape[0]
    oh_rows = (lab_rows == lax.broadcasted_iota(jnp.int32, (1, n_lab), 1)
               ).astype(jnp.float32)
    oh_cols = (lax.broadcasted_iota(jnp.int32, (n_lab, 1), 0) == lab_cols
               ).astype(jnp.float32)
    mrow = jnp.dot(oh_rows, mm_ref[...], preferred_element_type=jnp.float32)
    margins = jnp.dot(mrow, oh_cols, preferred_element_type=jnp.float32)

    cls_sz = jnp.sum(oh_cols, axis=1, keepdims=True)
    row_sz = jnp.dot(oh_rows, cls_sz, preferred_element_type=jnp.float32)
    has_pos = row_sz >= 2.0

    thr = d_ap + margins
    semi = (d2 > d2_ap_c) & (d2 < thr * thr)

    enc = (lax.bitcast_convert_type(d2, jnp.int32) & jnp.int32(~7)) | lab_cols
    enc_min = jnp.min(jnp.where(semi, enc, jnp.int32(_INF_BITS)),
                      axis=1, keepdims=True)

    d2ap_ref[pl.ds(r0, blk_r), :] = jnp.where(has_pos, d2_ap, neg_inf)
    enc_ref[pl.ds(r0, blk_r), :] = enc_min


def _newton_sqrt(x):
    bits = lax.bitcast_convert_type(x, jnp.int32)
    y = lax.bitcast_convert_type(
        jnp.int32(0x5F3759DF) - lax.shift_right_arithmetic(bits, 1),
        jnp.float32)
    for _ in range(3):
        y = y * (1.5 - 0.5 * x * y * y)
    return x * y


def _make_sc_stage(b, n_workers):
    apw = b // n_workers
    n_chunks = apw // 16
    mesh = plsc.VectorSubcoreMesh(core_axis_name="c", subcore_axis_name="s",
                                  num_cores=1, num_subcores=16)

    @functools.partial(
        pl.kernel, mesh=mesh,
        out_type=jax.ShapeDtypeStruct((16,), jnp.float32),
        scratch_types=[
            pltpu.VMEM((apw,), jnp.float32),
            pltpu.VMEM((apw,), jnp.int32),
            pltpu.VMEM((apw,), jnp.int32),
            pltpu.VMEM((64,), jnp.float32),
            pltpu.VMEM((16,), jnp.float32),
            pltpu.VMEM((16,), jnp.float32),
            pltpu.VMEM_SHARED((n_workers, 16), jnp.float32),
            pltpu.VMEM_SHARED((n_workers, 16), jnp.float32),
            pltpu.VMEM((n_workers, 16), jnp.float32),
            pltpu.VMEM((n_workers, 16), jnp.float32),
        ],
    )
    def sc_stage(d2ap_hbm, enc_hbm, lab_hbm, mm_hbm, out_hbm,
                 d2ap_v, enc_v, lab_v, mm_v, psum_v, pcnt_v,
                 sh_sum, sh_cnt, red_sum_v, red_cnt_v):
        wid = lax.axis_index("s")
        base = wid * apw
        pltpu.sync_copy(d2ap_hbm.at[pl.ds(base, apw)], d2ap_v)
        pltpu.sync_copy(enc_hbm.at[pl.ds(base, apw)], enc_v)
        pltpu.sync_copy(lab_hbm.at[pl.ds(base, apw)], lab_v)
        pltpu.sync_copy(mm_hbm, mm_v)

        t0 = mm_v[pl.ds(0, 16)]
        t1 = mm_v[pl.ds(16, 16)]
        t2 = mm_v[pl.ds(32, 16)]
        t3 = mm_v[pl.ds(48, 16)]

        psum = jnp.zeros((16,), jnp.float32)
        pcnt = jnp.zeros((16,), jnp.float32)
        for k in range(n_chunks):
            sl = pl.ds(k * 16, 16)
            d2ap_raw = d2ap_v[sl]
            enc = enc_v[sl]
            la = lab_v[sl]
            has_pos = d2ap_raw > jnp.float32(-jnp.inf)
            has_neg = enc < jnp.int32(_INF_BITS)
            lab_n = enc & jnp.int32(7)
            d2an = lax.bitcast_convert_type(enc & jnp.int32(~7), jnp.float32)
            d_ap = _newton_sqrt(jnp.maximum(d2ap_raw, 0.0))
            d_an = _newton_sqrt(d2an)
            midx = la * 8 + lab_n
            mlo = midx & jnp.int32(15)
            g0 = t0.at[mlo].get(mode="promise_in_bounds")
            g1 = t1.at[mlo].get(mode="promise_in_bounds")
            g2 = t2.at[mlo].get(mode="promise_in_bounds")
            g3 = t3.at[mlo].get(mode="promise_in_bounds")
            m = jnp.where(midx < 16, g0,
                          jnp.where(midx < 32, g1,
                                    jnp.where(midx < 48, g2, g3)))
            valid = has_pos & has_neg
            loss = jnp.maximum(d_ap - d_an + m, 0.0)
            psum = psum + jnp.where(valid, loss, 0.0)
            pcnt = pcnt + jnp.where(valid, 1.0, 0.0)
        psum_v[...] = psum
        pcnt_v[...] = pcnt
        pltpu.sync_copy(psum_v, sh_sum.at[wid])
        pltpu.sync_copy(pcnt_v, sh_cnt.at[wid])
        plsc.subcore_barrier()

        @pl.when(wid == 0)
        def _reduce():
            pltpu.sync_copy(sh_sum, red_sum_v)
            pltpu.sync_copy(sh_cnt, red_cnt_v)
            acc_s = red_sum_v[0, :]
            acc_c = red_cnt_v[0, :]
            for i in range(1, n_workers):
                acc_s = acc_s + red_sum_v[i, :]
                acc_c = acc_c + red_cnt_v[i, :]
            lanes = lax.broadcasted_iota(jnp.int32, (16,), 0)
            for sh in (8, 4, 2, 1):
                perm = lanes ^ jnp.int32(sh)
                acc_s = acc_s + acc_s.at[perm].get(mode="promise_in_bounds")
                acc_c = acc_c + acc_c.at[perm].get(mode="promise_in_bounds")
            res = jnp.where(acc_c > 0.0,
                            acc_s / jnp.maximum(acc_c, 1.0),
                            0.0)
            psum_v[...] = res
            pltpu.sync_copy(psum_v, out_hbm)

    return sc_stage


def kernel(embeddings, labels, margin_matrix):
    b, d = embeddings.shape
    blk_r = 512
    n_blk = b // blk_r
    n_workers = 16
    lab_row = labels.reshape(1, b)
    lab_col = labels.reshape(b, 1)
    n_lab = margin_matrix.shape[0]
    d2ap, enc = pl.pallas_call(
        functools.partial(_mine_kernel, blk_r=blk_r, b=b),
        grid=(n_blk,),
        in_specs=[
            pl.BlockSpec((b, d), lambda i: (0, 0)),
            pl.BlockSpec((1, b), lambda i: (0, 0)),
            pl.BlockSpec((b, 1), lambda i: (0, 0)),
            pl.BlockSpec((n_lab, n_lab), lambda i: (0, 0)),
        ],
        out_specs=[
            pl.BlockSpec((b, 1), lambda i: (0, 0)),
            pl.BlockSpec((b, 1), lambda i: (0, 0)),
        ],
        out_shape=[
            jax.ShapeDtypeStruct((b, 1), jnp.float32),
            jax.ShapeDtypeStruct((b, 1), jnp.int32),
        ],
        scratch_shapes=[
            pltpu.VMEM((b, d), jnp.float32),
            pltpu.VMEM((d, b), jnp.float32),
        ],
    )(embeddings, lab_row, lab_col, margin_matrix)

    sc_stage = _make_sc_stage(b, n_workers)
    out = sc_stage(d2ap.reshape(b), enc.reshape(b), labels,
                   margin_matrix.reshape(n_lab * n_lab))
    return out[0]
